# baseline (speedup 1.0000x reference)
import functools

import jax
import jax.numpy as jnp
from jax import lax
from jax.experimental import pallas as pl
from jax.experimental.pallas import tpu as pltpu
from jax.experimental.pallas import tpu_sc as plsc

_NEWTON_ITERS = 6


def _sparsemax_tau(x):
    rmax = jnp.max(x, axis=1, keepdims=True)
    lo = rmax - 1.0
    hi = rmax
    tau = lo
    for _ in range(_NEWTON_ITERS):
        pos = x > tau
        s = jnp.sum(jnp.where(pos, x, 0.0), axis=1, keepdims=True)
        cnt = jnp.sum(jnp.where(pos, 1.0, 0.0), axis=1, keepdims=True)
        g = s - cnt * tau - 1.0
        gpos = g > 0.0
        lo = jnp.where(gpos, tau, lo)
        hi = jnp.where(gpos, hi, tau)
        newt = tau + g / jnp.maximum(cnt, 1.0)
        mid = 0.5 * (lo + hi)
        take_newt = (newt > lo) & (newt < hi)
        tau = jnp.where(g == 0.0, tau, jnp.where(take_newt, newt, mid))
    return tau


def _sparsemax(x):
    return jnp.maximum(x - _sparsemax_tau(x), 0.0)


def _levels_body(conn2_ref, conn3_ref, root_ref, out_ref):
    level2 = jnp.dot(_sparsemax(conn3_ref[:]), root_ref[:],
                     preferred_element_type=jnp.float32)
    out_ref[:] = jnp.dot(_sparsemax(conn2_ref[:]), level2,
                         preferred_element_type=jnp.float32)


def _compute_level1(conn2, conn3, rootMatrix):
    n1, n2 = conn2.shape
    d = rootMatrix.shape[1]
    return pl.pallas_call(
        _levels_body,
        out_shape=jax.ShapeDtypeStruct((n1, d), jnp.float32),
    )(conn2, conn3, rootMatrix)


def _main_body(rows_ref, l1_ref, out_ref):
    x = rows_ref[:]
    sm = jnp.maximum(x - _sparsemax_tau(x), 0.0)
    out_ref[:] = jnp.dot(sm, l1_ref[:], preferred_element_type=jnp.float32)


def _sc_gather(table, idx):
    batch = idx.shape[0]
    depth = table.shape[1]
    num_cores, num_subcores = 2, 16
    num_workers = num_cores * num_subcores
    per_worker = batch // num_workers
    chunk = 32 if per_worker % 32 == 0 else per_worker
    n_ch = per_worker // chunk

    mesh = plsc.VectorSubcoreMesh(core_axis_name="c", subcore_axis_name="s")

    @functools.partial(
        pl.kernel,
        mesh=mesh,
        out_type=jax.ShapeDtypeStruct((batch, depth), jnp.float32),
        scratch_types=[
            pltpu.VMEM((per_worker,), jnp.int32),
            pltpu.VMEM((chunk, depth), jnp.float32),
            pltpu.VMEM((chunk, depth), jnp.float32),
            pltpu.SemaphoreType.DMA,
            pltpu.SemaphoreType.DMA,
            pltpu.SemaphoreType.DMA,
            pltpu.SemaphoreType.DMA,
        ],
    )
    def gather_kernel(table_hbm, idx_hbm, out_hbm,
                      idx_v, buf0, buf1, gs0, gs1, ws0, ws1):
        wid = lax.axis_index("s") * num_cores + lax.axis_index("c")
        base = wid * per_worker
        pltpu.sync_copy(idx_hbm.at[pl.ds(base, per_worker)], idx_v)
        bufs, gsems, wsems = (buf0, buf1), (gs0, gs1), (ws0, ws1)
        gd = [None] * n_ch
        wd = [None] * n_ch
        for c in range(n_ch):
            b = c & 1
            if c >= 2:
                wd[c - 2].wait()
            gd[c] = pltpu.async_copy(
                table_hbm.at[idx_v.at[pl.ds(c * chunk, chunk)]],
                bufs[b], gsems[b])
            if c >= 1:
                gd[c - 1].wait()
                wd[c - 1] = pltpu.async_copy(
                    bufs[(c - 1) & 1],
                    out_hbm.at[pl.ds(base + (c - 1) * chunk, chunk)],
                    wsems[(c - 1) & 1])
        gd[n_ch - 1].wait()
        wd[n_ch - 1] = pltpu.async_copy(
            bufs[(n_ch - 1) & 1],
            out_hbm.at[pl.ds(base + (n_ch - 1) * chunk, chunk)],
            wsems[(n_ch - 1) & 1])
        if n_ch >= 2:
            wd[n_ch - 2].wait()
        wd[n_ch - 1].wait()

    return gather_kernel(table, idx)


def _main_body_first(rows_ref, l1_ref, out_ref):
    _main_body(rows_ref, l1_ref, out_ref)


def _main_body_carry(rows_ref, l1_ref, carry_ref, out_ref):
    del carry_ref
    _main_body(rows_ref, l1_ref, out_ref)


def kernel(ids, conn1, conn2, conn3, rootMatrix):
    batch = ids.shape[0]
    k_dim = conn1.shape[1]
    d = rootMatrix.shape[1]

    level1 = _compute_level1(conn2, conn3, rootMatrix)
    ids32 = ids.astype(jnp.int32)

    n_chunks = 4
    chunk_b = batch // n_chunks
    block_rows = 1024
    blocks_per_chunk = chunk_b // block_rows

    out = None
    for c in range(n_chunks):
        ids_c = ids32[c * chunk_b:(c + 1) * chunk_b]
        rows = _sc_gather(conn1, ids_c)
        row_spec = pl.BlockSpec((block_rows, k_dim), lambda i: (i, 0))
        l1_spec = pl.BlockSpec((k_dim, d), lambda i: (0, 0))
        out_spec = pl.BlockSpec(
            (block_rows, d), lambda i, c=c: (c * blocks_per_chunk + i, 0))
        if c == 0:
            out = pl.pallas_call(
                _main_body_first,
                grid=(blocks_per_chunk,),
                in_specs=[row_spec, l1_spec],
                out_specs=out_spec,
                out_shape=jax.ShapeDtypeStruct((batch, d), jnp.float32),
            )(rows, level1)
        else:
            out = pl.pallas_call(
                _main_body_carry,
                grid=(blocks_per_chunk,),
                in_specs=[row_spec, l1_spec,
                          pl.BlockSpec(memory_space=pltpu.MemorySpace.HBM)],
                out_specs=out_spec,
                out_shape=jax.ShapeDtypeStruct((batch, d), jnp.float32),
                input_output_aliases={2: 0},
            )(rows, level1, out)
    return out


# speedup vs baseline: 1.0227x; 1.0227x over previous
import functools

import jax
import jax.numpy as jnp
from jax import lax
from jax.experimental import pallas as pl
from jax.experimental.pallas import tpu as pltpu
from jax.experimental.pallas import tpu_sc as plsc

_NEWTON_ITERS = 6


def _sparsemax_tau(x):
    rmax = jnp.max(x, axis=1, keepdims=True)
    lo = rmax - 1.0
    hi = rmax
    tau = lo
    for _ in range(_NEWTON_ITERS):
        pos = x > tau
        s = jnp.sum(jnp.where(pos, x, 0.0), axis=1, keepdims=True)
        cnt = jnp.sum(jnp.where(pos, 1.0, 0.0), axis=1, keepdims=True)
        g = s - cnt * tau - 1.0
        gpos = g > 0.0
        lo = jnp.where(gpos, tau, lo)
        hi = jnp.where(gpos, hi, tau)
        newt = tau + g / jnp.maximum(cnt, 1.0)
        mid = 0.5 * (lo + hi)
        take_newt = (newt > lo) & (newt < hi)
        tau = jnp.where(g == 0.0, tau, jnp.where(take_newt, newt, mid))
    return tau


def _sparsemax(x):
    return jnp.maximum(x - _sparsemax_tau(x), 0.0)


def _levels_body(conn2_ref, conn3_ref, root_ref, out_ref):
    level2 = jnp.dot(_sparsemax(conn3_ref[:]), root_ref[:],
                     preferred_element_type=jnp.float32)
    out_ref[:] = jnp.dot(_sparsemax(conn2_ref[:]), level2,
                         preferred_element_type=jnp.float32)


def _compute_level1(conn2, conn3, rootMatrix):
    n1, n2 = conn2.shape
    d = rootMatrix.shape[1]
    return pl.pallas_call(
        _levels_body,
        out_shape=jax.ShapeDtypeStruct((n1, d), jnp.float32),
    )(conn2, conn3, rootMatrix)


def _main_body(rows_ref, l1_ref, out_ref):
    x = rows_ref[:]
    sm = jnp.maximum(x - _sparsemax_tau(x), 0.0)
    out_ref[:] = jnp.dot(sm, l1_ref[:], preferred_element_type=jnp.float32)


def _sc_gather(table, idx):
    batch = idx.shape[0]
    depth = table.shape[1]
    num_cores, num_subcores = 2, 16
    num_workers = num_cores * num_subcores
    per_worker = batch // num_workers
    chunk = 32 if per_worker % 32 == 0 else per_worker
    n_ch = per_worker // chunk

    mesh = plsc.VectorSubcoreMesh(core_axis_name="c", subcore_axis_name="s")

    @functools.partial(
        pl.kernel,
        mesh=mesh,
        out_type=jax.ShapeDtypeStruct((batch, depth), jnp.float32),
        scratch_types=[
            pltpu.VMEM((per_worker,), jnp.int32),
            pltpu.VMEM((chunk, depth), jnp.float32),
            pltpu.VMEM((chunk, depth), jnp.float32),
            pltpu.SemaphoreType.DMA,
            pltpu.SemaphoreType.DMA,
            pltpu.SemaphoreType.DMA,
            pltpu.SemaphoreType.DMA,
        ],
    )
    def gather_kernel(table_hbm, idx_hbm, out_hbm,
                      idx_v, buf0, buf1, gs0, gs1, ws0, ws1):
        wid = lax.axis_index("s") * num_cores + lax.axis_index("c")
        base = wid * per_worker
        pltpu.sync_copy(idx_hbm.at[pl.ds(base, per_worker)], idx_v)
        bufs, gsems, wsems = (buf0, buf1), (gs0, gs1), (ws0, ws1)
        gd = [None] * n_ch
        wd = [None] * n_ch
        for c in range(n_ch):
            b = c & 1
            if c >= 2:
                wd[c - 2].wait()
            gd[c] = pltpu.async_copy(
                table_hbm.at[idx_v.at[pl.ds(c * chunk, chunk)]],
                bufs[b], gsems[b])
            if c >= 1:
                gd[c - 1].wait()
                wd[c - 1] = pltpu.async_copy(
                    bufs[(c - 1) & 1],
                    out_hbm.at[pl.ds(base + (c - 1) * chunk, chunk)],
                    wsems[(c - 1) & 1])
        gd[n_ch - 1].wait()
        wd[n_ch - 1] = pltpu.async_copy(
            bufs[(n_ch - 1) & 1],
            out_hbm.at[pl.ds(base + (n_ch - 1) * chunk, chunk)],
            wsems[(n_ch - 1) & 1])
        if n_ch >= 2:
            wd[n_ch - 2].wait()
        wd[n_ch - 1].wait()

    return gather_kernel(table, idx)


def _main_body_first(rows_ref, l1_ref, out_ref):
    _main_body(rows_ref, l1_ref, out_ref)


def _main_body_carry(rows_ref, l1_ref, carry_ref, out_ref):
    del carry_ref
    _main_body(rows_ref, l1_ref, out_ref)


def kernel(ids, conn1, conn2, conn3, rootMatrix):
    batch = ids.shape[0]
    k_dim = conn1.shape[1]
    d = rootMatrix.shape[1]

    level1 = _compute_level1(conn2, conn3, rootMatrix)
    ids32 = ids.astype(jnp.int32)

    if batch % 16384 == 0:
        chunk_sizes = [2048, 4096, 5120, 5120] * (batch // 16384)
    else:
        chunk_sizes = [batch // 4] * 4
    block_rows = 512

    out = None
    offset = 0
    for c, chunk_b in enumerate(chunk_sizes):
        blocks_per_chunk = chunk_b // block_rows
        block_off = offset // block_rows
        ids_c = ids32[offset:offset + chunk_b]
        offset += chunk_b
        rows = _sc_gather(conn1, ids_c)
        row_spec = pl.BlockSpec((block_rows, k_dim), lambda i: (i, 0))
        l1_spec = pl.BlockSpec((k_dim, d), lambda i: (0, 0))
        out_spec = pl.BlockSpec(
            (block_rows, d), lambda i, o=block_off: (o + i, 0))
        if c == 0:
            out = pl.pallas_call(
                _main_body_first,
                grid=(blocks_per_chunk,),
                in_specs=[row_spec, l1_spec],
                out_specs=out_spec,
                out_shape=jax.ShapeDtypeStruct((batch, d), jnp.float32),
            )(rows, level1)
        else:
            out = pl.pallas_call(
                _main_body_carry,
                grid=(blocks_per_chunk,),
                in_specs=[row_spec, l1_spec,
                          pl.BlockSpec(memory_space=pltpu.MemorySpace.HBM)],
                out_specs=out_spec,
                out_shape=jax.ShapeDtypeStruct((batch, d), jnp.float32),
                input_output_aliases={2: 0},
            )(rows, level1, out)
    return out


# speedup vs baseline: 1.1104x; 1.0858x over previous
import functools

import jax
import jax.numpy as jnp
from jax import lax
from jax.experimental import pallas as pl
from jax.experimental.pallas import tpu as pltpu
from jax.experimental.pallas import tpu_sc as plsc

_NEWTON_ITERS = 6


def _sparsemax_tau(x):
    rmax = jnp.max(x, axis=1, keepdims=True)
    lo = rmax - 1.0
    hi = rmax
    tau = lo
    for _ in range(_NEWTON_ITERS):
        pos = x > tau
        s = jnp.sum(jnp.where(pos, x, 0.0), axis=1, keepdims=True)
        cnt = jnp.sum(jnp.where(pos, 1.0, 0.0), axis=1, keepdims=True)
        g = s - cnt * tau - 1.0
        gpos = g > 0.0
        lo = jnp.where(gpos, tau, lo)
        hi = jnp.where(gpos, hi, tau)
        newt = tau + g / jnp.maximum(cnt, 1.0)
        mid = 0.5 * (lo + hi)
        take_newt = (newt > lo) & (newt < hi)
        tau = jnp.where(g == 0.0, tau, jnp.where(take_newt, newt, mid))
    return tau


def _sparsemax(x):
    return jnp.maximum(x - _sparsemax_tau(x), 0.0)


def _levels_body(conn2_ref, conn3_ref, root_ref, out_ref):
    level2 = jnp.dot(_sparsemax(conn3_ref[:]), root_ref[:],
                     preferred_element_type=jnp.float32)
    out_ref[:] = jnp.dot(_sparsemax(conn2_ref[:]), level2,
                         preferred_element_type=jnp.float32)


def _compute_level1(conn2, conn3, rootMatrix):
    n1, n2 = conn2.shape
    d = rootMatrix.shape[1]
    return pl.pallas_call(
        _levels_body,
        out_shape=jax.ShapeDtypeStruct((n1, d), jnp.float32),
    )(conn2, conn3, rootMatrix)


def _sparsemax_tau_hybrid(x):
    rmax = jnp.max(x, axis=1, keepdims=True)
    lo = rmax - 1.0
    hi = rmax
    xb = x.astype(jnp.bfloat16)
    tau = lo
    for _ in range(3):
        tb = tau.astype(jnp.bfloat16)
        pos = xb > tb
        s = jnp.sum(jnp.where(pos, xb, jnp.bfloat16(0.0)), axis=1,
                    keepdims=True, dtype=jnp.bfloat16).astype(jnp.float32)
        cnt = jnp.sum(jnp.where(pos, jnp.bfloat16(1.0), jnp.bfloat16(0.0)),
                      axis=1, keepdims=True,
                      dtype=jnp.bfloat16).astype(jnp.float32)
        g = s - cnt * tau - 1.0
        gpos = g > 0.0
        lo = jnp.where(gpos, tau, lo)
        hi = jnp.where(gpos, hi, tau)
        newt = tau + g / jnp.maximum(cnt, 1.0)
        mid = 0.5 * (lo + hi)
        take_newt = (newt > lo) & (newt < hi)
        tau = jnp.where(take_newt, newt, mid)
    lo = rmax - 1.0
    hi = rmax
    tau = jnp.maximum(tau - 0.06, lo)
    for _ in range(3):
        pos = x > tau
        s = jnp.sum(jnp.where(pos, x, 0.0), axis=1, keepdims=True)
        cnt = jnp.sum(jnp.where(pos, 1.0, 0.0), axis=1, keepdims=True)
        g = s - cnt * tau - 1.0
        gpos = g > 0.0
        lo = jnp.where(gpos, tau, lo)
        hi = jnp.where(gpos, hi, tau)
        newt = tau + g / jnp.maximum(cnt, 1.0)
        mid = 0.5 * (lo + hi)
        take_newt = (newt > lo) & (newt < hi)
        tau = jnp.where(g == 0.0, tau, jnp.where(take_newt, newt, mid))
    return tau


def _main_body(rows_ref, l1_ref, out_ref):
    x = rows_ref[:]
    sm = jnp.maximum(x - _sparsemax_tau_hybrid(x), 0.0)
    out_ref[:] = jnp.dot(sm, l1_ref[:], preferred_element_type=jnp.float32)


def _sc_gather(table, idx):
    batch = idx.shape[0]
    depth = table.shape[1]
    num_cores, num_subcores = 2, 16
    num_workers = num_cores * num_subcores
    per_worker = batch // num_workers
    chunk = 32 if per_worker % 32 == 0 else per_worker
    n_ch = per_worker // chunk

    mesh = plsc.VectorSubcoreMesh(core_axis_name="c", subcore_axis_name="s")

    @functools.partial(
        pl.kernel,
        mesh=mesh,
        out_type=jax.ShapeDtypeStruct((batch, depth), jnp.float32),
        scratch_types=[
            pltpu.VMEM((per_worker,), jnp.int32),
            pltpu.VMEM((chunk, depth), jnp.float32),
            pltpu.VMEM((chunk, depth), jnp.float32),
            pltpu.SemaphoreType.DMA,
            pltpu.SemaphoreType.DMA,
            pltpu.SemaphoreType.DMA,
            pltpu.SemaphoreType.DMA,
        ],
    )
    def gather_kernel(table_hbm, idx_hbm, out_hbm,
                      idx_v, buf0, buf1, gs0, gs1, ws0, ws1):
        wid = lax.axis_index("s") * num_cores + lax.axis_index("c")
        base = wid * per_worker
        pltpu.sync_copy(idx_hbm.at[pl.ds(base, per_worker)], idx_v)
        bufs, gsems, wsems = (buf0, buf1), (gs0, gs1), (ws0, ws1)
        gd = [None] * n_ch
        wd = [None] * n_ch
        for c in range(n_ch):
            b = c & 1
            if c >= 2:
                wd[c - 2].wait()
            gd[c] = pltpu.async_copy(
                table_hbm.at[idx_v.at[pl.ds(c * chunk, chunk)]],
                bufs[b], gsems[b])
            if c >= 1:
                gd[c - 1].wait()
                wd[c - 1] = pltpu.async_copy(
                    bufs[(c - 1) & 1],
                    out_hbm.at[pl.ds(base + (c - 1) * chunk, chunk)],
                    wsems[(c - 1) & 1])
        gd[n_ch - 1].wait()
        wd[n_ch - 1] = pltpu.async_copy(
            bufs[(n_ch - 1) & 1],
            out_hbm.at[pl.ds(base + (n_ch - 1) * chunk, chunk)],
            wsems[(n_ch - 1) & 1])
        if n_ch >= 2:
            wd[n_ch - 2].wait()
        wd[n_ch - 1].wait()

    return gather_kernel(table, idx)


def _main_body_first(rows_ref, l1_ref, out_ref):
    _main_body(rows_ref, l1_ref, out_ref)


def _main_body_carry(rows_ref, l1_ref, carry_ref, out_ref):
    del carry_ref
    _main_body(rows_ref, l1_ref, out_ref)


def kernel(ids, conn1, conn2, conn3, rootMatrix):
    batch = ids.shape[0]
    k_dim = conn1.shape[1]
    d = rootMatrix.shape[1]

    level1 = _compute_level1(conn2, conn3, rootMatrix)
    ids32 = ids.astype(jnp.int32)

    if batch % 16384 == 0:
        chunk_sizes = [2048, 4096, 5120, 5120] * (batch // 16384)
    else:
        chunk_sizes = [batch // 4] * 4
    block_rows = 512

    out = None
    offset = 0
    for c, chunk_b in enumerate(chunk_sizes):
        blocks_per_chunk = chunk_b // block_rows
        block_off = offset // block_rows
        ids_c = ids32[offset:offset + chunk_b]
        offset += chunk_b
        rows = _sc_gather(conn1, ids_c)
        row_spec = pl.BlockSpec((block_rows, k_dim), lambda i: (i, 0))
        l1_spec = pl.BlockSpec((k_dim, d), lambda i: (0, 0))
        out_spec = pl.BlockSpec(
            (block_rows, d), lambda i, o=block_off: (o + i, 0))
        if c == 0:
            out = pl.pallas_call(
                _main_body_first,
                grid=(blocks_per_chunk,),
                in_specs=[row_spec, l1_spec],
                out_specs=out_spec,
                out_shape=jax.ShapeDtypeStruct((batch, d), jnp.float32),
            )(rows, level1)
        else:
            out = pl.pallas_call(
                _main_body_carry,
                grid=(blocks_per_chunk,),
                in_specs=[row_spec, l1_spec,
                          pl.BlockSpec(memory_space=pltpu.MemorySpace.HBM)],
                out_specs=out_spec,
                out_shape=jax.ShapeDtypeStruct((batch, d), jnp.float32),
                input_output_aliases={2: 0},
            )(rows, level1, out)
    return out
